# baseline (speedup 1.0000x reference)
import jax
import jax.numpy as jnp
from jax import lax
from jax.experimental import pallas as pl
from jax.experimental.pallas import tpu as pltpu
from jax.experimental.pallas import tpu_sc as plsc

B, N_ITEMS, N_SETS, K_SET, D = 256, 2000, 500, 50, 16

NC, NS, L = 2, 16, 16
NW = NC * NS
ROWS_PER_W = B // NW
NBLK = (N_SETS + L - 1) // L
S_PAD = NBLK * L
M_LEN = N_SETS * K_SET
M_PAD = S_PAD * K_SET
TC_RB = 8


def _sc_body(w_hbm, m_hbm, g_hbm,
             w_v0, w_v1, m_v0, m_v1, g_v0, g_v1,
             sem_w, sem_m, sem_g):
    cid = lax.axis_index("c")
    sid = lax.axis_index("s")
    wid = sid * NC + cid

    lane = lax.iota(jnp.int32, L)
    lane50 = lane * K_SET

    z16 = jnp.zeros((L,), jnp.int32)
    for m_v in (m_v0, m_v1):
        for t in range(M_LEN - (M_LEN % L), M_PAD, L):
            m_v[pl.ds(t, L)] = z16

    w_bufs = (w_v0, w_v1)
    m_bufs = (m_v0, m_v1)
    g_bufs = (g_v0, g_v1)

    def row_of(r):
        return wid * ROWS_PER_W + r

    def issue_in(r):
        b = row_of(r)
        dw = pltpu.async_copy(
            w_hbm.at[pl.ds(b * N_ITEMS, N_ITEMS)], w_bufs[r % 2], sem_w)
        dm = pltpu.async_copy(
            m_hbm.at[pl.ds(b * M_LEN, M_LEN)],
            m_bufs[r % 2].at[pl.ds(0, M_LEN)], sem_m)
        return dw, dm

    in_descs = {0: issue_in(0)}
    out_descs = [None, None]

    for r in range(ROWS_PER_W):
        w_v, m_v, g_v = w_bufs[r % 2], m_bufs[r % 2], g_bufs[r % 2]
        dw, dm = in_descs.pop(r)
        dw.wait()
        dm.wait()
        if r + 1 < ROWS_PER_W:
            in_descs[r + 1] = issue_in(r + 1)
        if out_descs[r % 2] is not None:
            out_descs[r % 2].wait()

        def blk_body(i):
            addr0 = lane50 + i * (L * K_SET)
            acc0 = jnp.zeros((L,), jnp.float32)
            acc1 = jnp.zeros((L,), jnp.float32)
            for k in range(0, K_SET, 2):
                iv0 = plsc.load_gather(m_v, [addr0 + k])
                iv1 = plsc.load_gather(m_v, [addr0 + (k + 1)])
                acc0 = acc0 + plsc.load_gather(w_v, [iv0])
                acc1 = acc1 + plsc.load_gather(w_v, [iv1])
            g_v[pl.ds(i * L, L)] = acc0 + acc1

        plsc.parallel_loop(0, NBLK, 1, unroll=1)(blk_body)

        out_descs[r % 2] = pltpu.async_copy(
            g_v, g_hbm.at[pl.ds(row_of(r) * S_PAD, S_PAD)], sem_g)

    out_descs[0].wait()
    out_descs[1].wait()


def _tc_body(g_ref, wv_ref, b_ref, o_ref):
    wv = wv_ref[...]
    kb = jnp.float32(K_SET) * b_ref[...]
    for r in range(TC_RB):
        g = g_ref[r][None]
        res = lax.dot_general(
            g[:, :N_SETS], wv, (((0,), (0,)), ((), ())),
            precision=lax.Precision.HIGHEST,
            preferred_element_type=jnp.float32)
        o_ref[r] = res + kb


@jax.jit
def _mcp_embed(weights, memb_flat, wv, bias):
    run = pl.kernel(
        _sc_body,
        out_type=jax.ShapeDtypeStruct((B * S_PAD,), jnp.float32),
        mesh=plsc.VectorSubcoreMesh(core_axis_name="c", subcore_axis_name="s"),
        scratch_types=[
            pltpu.VMEM((N_ITEMS,), jnp.float32),
            pltpu.VMEM((N_ITEMS,), jnp.float32),
            pltpu.VMEM((M_PAD,), jnp.int32),
            pltpu.VMEM((M_PAD,), jnp.int32),
            pltpu.VMEM((S_PAD,), jnp.float32),
            pltpu.VMEM((S_PAD,), jnp.float32),
            pltpu.SemaphoreType.DMA,
            pltpu.SemaphoreType.DMA,
            pltpu.SemaphoreType.DMA,
        ],
        compiler_params=pltpu.CompilerParams(needs_layout_passes=False),
    )
    gsum = run(weights, memb_flat).reshape(B, S_PAD)

    return pl.pallas_call(
        _tc_body,
        grid=(B // TC_RB,),
        in_specs=[
            pl.BlockSpec((TC_RB, S_PAD), lambda i: (i, 0)),
            pl.BlockSpec((1, D), lambda i: (0, 0)),
            pl.BlockSpec((1, D), lambda i: (0, 0)),
        ],
        out_specs=pl.BlockSpec((TC_RB, N_SETS, D), lambda i: (i, 0, 0)),
        out_shape=jax.ShapeDtypeStruct((B, N_SETS, D), jnp.float32),
    )(gsum, wv, bias)


def kernel(weights, membership, W, b):
    memb_flat = membership.astype(jnp.int32).reshape(B * M_LEN)
    return _mcp_embed(weights.reshape(B * N_ITEMS), memb_flat,
                      W[:, 0].reshape(1, D), b.reshape(1, D))
